# baseline (speedup 1.0000x reference)
import functools

import jax
import jax.numpy as jnp
from jax.experimental import pallas as pl

N_NODES = 50000
DIM = 64


def _norm_body(v_ref, o_ref):
    v = v_ref[...]
    n = jnp.sqrt(jnp.sum(v * v, axis=-1, keepdims=True))
    o_ref[...] = v / jnp.maximum(n, 1e-12)


def _normalize(node_vectors):
    return pl.pallas_call(
        _norm_body,
        grid=(50,),
        in_specs=[pl.BlockSpec((1000, DIM), lambda i: (i, 0))],
        out_specs=pl.BlockSpec((1000, DIM), lambda i: (i, 0)),
        out_shape=jax.ShapeDtypeStruct((N_NODES, DIM), jnp.float32),
    )(node_vectors)


def kernel(node_vectors, edge_index, edges):
    x = _normalize(node_vectors)

    row = edge_index[0]
    col = edge_index[1]

    one_hop_x = jax.ops.segment_sum(x[col], row, num_segments=N_NODES)
    two_hop_x = jax.ops.segment_sum(one_hop_x[col], row, num_segments=N_NODES)
    degree_one_hop = jax.ops.segment_sum(
        jnp.ones(row.shape, dtype=x.dtype), row, num_segments=N_NODES)

    e0 = edges[0]
    e1 = edges[1]

    def dot(a, b):
        return (a * b).sum(axis=-1)

    count_1_1 = dot(one_hop_x[e0], one_hop_x[e1])
    count_1_2 = dot(one_hop_x[e0], two_hop_x[e1]) + dot(two_hop_x[e0], one_hop_x[e1])
    count_2_2 = dot(two_hop_x[e0] - degree_one_hop[e0][:, None] * x[e0],
                    two_hop_x[e1] - degree_one_hop[e1][:, None] * x[e1])
    count_self_1_2 = dot(one_hop_x[e0], two_hop_x[e0]) + dot(one_hop_x[e1], two_hop_x[e1])
    return (count_1_1, count_1_2, count_2_2, count_self_1_2)


# speedup vs baseline: 3.7867x; 3.7867x over previous
import functools

import jax
import jax.numpy as jnp
from jax import lax
from jax.experimental import pallas as pl
from jax.experimental.pallas import tpu as pltpu
from jax.experimental.pallas import tpu_sc as plsc

N = 50000
D = 64
E = 800000
Q = 16384
QF = 2 * Q

NC = 2
NS = 16
NW = NC * NS

HALF = N // 2
ACC_ROWS = HALF + NS + 8
K = 128
CHUNKS = 392
EPAD = NS * CHUNKS * K
ZR = 1000
NZCH = 26
DZ = 1024
NZD = 25
NWD = 25

QPW = QF // NW
QCH = QPW // K


def _mesh():
    return plsc.VectorSubcoreMesh(
        core_axis_name="c", subcore_axis_name="s",
        num_cores=NC, num_subcores=NS)


def _norm_body(v_ref, o_ref):
    v = v_ref[...]
    n = jnp.sqrt(jnp.sum(v * v, axis=-1, keepdims=True))
    o_ref[...] = v / jnp.maximum(n, 1e-12)


def _normalize(node_vectors):
    return pl.pallas_call(
        _norm_body,
        grid=(50,),
        in_specs=[pl.BlockSpec((1000, D), lambda i: (i, 0))],
        out_specs=pl.BlockSpec((1000, D), lambda i: (i, 0)),
        out_shape=jax.ShapeDtypeStruct((N, D), jnp.float32),
    )(node_vectors)


def _make_spmm(with_degree):
    out_type = [jax.ShapeDtypeStruct((N, D), jnp.float32)]
    if with_degree:
        out_type.append(jax.ShapeDtypeStruct((N,), jnp.float32))
    scratch = [
        pltpu.VMEM_SHARED((ACC_ROWS, D), jnp.float32),
        pltpu.VMEM((K,), jnp.int32),
        pltpu.VMEM((K,), jnp.int32),
        pltpu.VMEM((K, D), jnp.float32),
        pltpu.SemaphoreType.DMA,
        pltpu.SemaphoreType.DMA,
    ]
    if with_degree:
        scratch += [
            pltpu.VMEM_SHARED((ACC_ROWS,), jnp.float32),
            pltpu.VMEM((K,), jnp.float32),
            pltpu.VMEM((DZ,), jnp.float32),
        ]

    def body(x_hbm, row_hbm, col_hbm, z2_hbm, *rest):
        if with_degree:
            (out_hbm, deg_hbm, acc_sh, col_v, loc_v, rows_v, gsem, ssem,
             deg_sh, ones_v, stage_v) = rest
        else:
            out_hbm, acc_sh, col_v, loc_v, rows_v, gsem, ssem = rest
        c = lax.axis_index("c")
        s = lax.axis_index("s")

        if with_degree:
            for i in range(DZ // 16):
                stage_v[pl.ds(i * 16, 16)] = jnp.zeros((16,), jnp.float32)
        for j in range(NZCH):
            size = ZR if j < NZCH - 1 else ACC_ROWS - (NZCH - 1) * ZR

            @pl.when(s == j % NS)
            def _zero(j=j, size=size):
                pltpu.sync_copy(z2_hbm.at[pl.ds(0, size)],
                                acc_sh.at[pl.ds(j * ZR, size)])

        if with_degree:
            for j in range(NZD):
                size = DZ if j < NZD - 1 else ACC_ROWS - (NZD - 1) * DZ

                @pl.when(s == j % NS)
                def _zerod(j=j, size=size):
                    pltpu.sync_copy(stage_v.at[pl.ds(0, size)],
                                    deg_sh.at[pl.ds(j * DZ, size)])
            for i in range(K // 16):
                ones_v[pl.ds(i * 16, 16)] = jnp.full((16,), 1.0, jnp.float32)
        plsc.subcore_barrier()

        base = s * (CHUNKS * K)
        lo = c * HALF
        dummy = HALF + s

        def chunk(j, carry):
            off = base + j * K
            pltpu.sync_copy(col_hbm.at[pl.ds(off, K)], col_v)
            gd = pltpu.async_copy(x_hbm.at[col_v], rows_v, gsem)
            pltpu.sync_copy(row_hbm.at[pl.ds(off, K)], loc_v)
            for i in range(K // 16):
                rv = loc_v[pl.ds(i * 16, 16)]
                lv = rv - lo
                ok = (lv >= 0) & (lv < HALF)
                loc_v[pl.ds(i * 16, 16)] = jnp.where(ok, lv, dummy)
            gd.wait()
            pltpu.async_copy(rows_v, acc_sh.at[loc_v], ssem, add=True).wait()
            if with_degree:
                pltpu.async_copy(ones_v, deg_sh.at[loc_v], ssem,
                                 add=True).wait()
            return carry

        lax.fori_loop(0, CHUNKS, chunk, 0)
        plsc.subcore_barrier()

        for j in range(25):
            @pl.when(s == j % NS)
            def _wb(j=j):
                pltpu.sync_copy(acc_sh.at[pl.ds(j * ZR, ZR)],
                                out_hbm.at[pl.ds(c * HALF + j * ZR, ZR)])

        if with_degree:
            for j in range(NWD):
                size = DZ if j < NWD - 1 else HALF - (NWD - 1) * DZ

                @pl.when(s == j % NS)
                def _wbd(j=j, size=size):
                    pltpu.sync_copy(deg_sh.at[pl.ds(j * DZ, size)],
                                    stage_v.at[pl.ds(0, size)])
                    pltpu.sync_copy(stage_v.at[pl.ds(0, size)],
                                    deg_hbm.at[pl.ds(c * HALF + j * DZ, size)])

    return pl.kernel(
        body,
        out_type=tuple(out_type),
        mesh=_mesh(),
        scratch_types=scratch,
        compiler_params=pltpu.CompilerParams(use_tc_tiling_on_sc=False),
    )


def _make_qgather():
    sds = jax.ShapeDtypeStruct
    scratch = [
        pltpu.VMEM((K,), jnp.int32),
        pltpu.VMEM((K, D), jnp.float32),
        pltpu.VMEM((K, D), jnp.float32),
        pltpu.VMEM((K, D), jnp.float32),
        pltpu.VMEM((K,), jnp.float32),
        pltpu.SemaphoreType.DMA,
    ]

    def body(x_hbm, h1_hbm, h2_hbm, deg_hbm, ef_hbm,
             ox, o1, o2, od, idx_v, bx, b1, b2, bd, sem):
        c = lax.axis_index("c")
        s = lax.axis_index("s")
        wid = s * NC + c
        base = wid * QPW

        def chunk(j, carry):
            off = base + j * K
            pltpu.sync_copy(ef_hbm.at[pl.ds(off, K)], idx_v)
            d1 = pltpu.async_copy(x_hbm.at[idx_v], bx, sem)
            d2 = pltpu.async_copy(h1_hbm.at[idx_v], b1, sem)
            d3 = pltpu.async_copy(h2_hbm.at[idx_v], b2, sem)
            d4 = pltpu.async_copy(deg_hbm.at[idx_v], bd, sem)
            d1.wait(); d2.wait(); d3.wait(); d4.wait()
            pltpu.sync_copy(bx, ox.at[pl.ds(off, K)])
            pltpu.sync_copy(b1, o1.at[pl.ds(off, K)])
            pltpu.sync_copy(b2, o2.at[pl.ds(off, K)])
            pltpu.sync_copy(bd, od.at[pl.ds(off, K)])
            return carry

        lax.fori_loop(0, QCH, chunk, 0)

    return pl.kernel(
        body,
        out_type=(sds((QF, D), jnp.float32), sds((QF, D), jnp.float32),
                  sds((QF, D), jnp.float32), sds((QF,), jnp.float32)),
        mesh=_mesh(),
        scratch_types=scratch,
        compiler_params=pltpu.CompilerParams(use_tc_tiling_on_sc=False),
    )


def _decode_body(x0, x1, h10, h11, h20, h21, d0, d1, o11, o12, o22, os12):
    X0 = x0[...]; X1 = x1[...]
    A0 = h10[...]; A1 = h11[...]
    B0 = h20[...]; B1 = h21[...]
    t0 = B0 - d0[...] * X0
    t1 = B1 - d1[...] * X1

    def dot(a, b):
        return jnp.sum(a * b, axis=-1, keepdims=True)

    o11[...] = dot(A0, A1)
    o12[...] = dot(A0, B1) + dot(B0, A1)
    o22[...] = dot(t0, t1)
    os12[...] = dot(A0, B0) + dot(A1, B1)


def _decode(x0, x1, h10, h11, h20, h21, d0, d1):
    B = 2048
    mat = pl.BlockSpec((B, D), lambda i: (i, 0))
    vec = pl.BlockSpec((B, 1), lambda i: (i, 0))
    sds = jax.ShapeDtypeStruct
    return pl.pallas_call(
        _decode_body,
        grid=(Q // B,),
        in_specs=[mat] * 6 + [vec] * 2,
        out_specs=[vec] * 4,
        out_shape=[sds((Q, 1), jnp.float32)] * 4,
    )(x0, x1, h10, h11, h20, h21, d0, d1)


_spmm_deg = _make_spmm(True)
_spmm = _make_spmm(False)
_qgather = _make_qgather()


def kernel(node_vectors, edge_index, edges):
    x = _normalize(node_vectors.astype(jnp.float32))
    ei = edge_index.astype(jnp.int32)
    row = jnp.concatenate([ei[0], jnp.full((EPAD - E,), -1, jnp.int32)])
    col = jnp.concatenate([ei[1], jnp.zeros((EPAD - E,), jnp.int32)])
    z2 = jnp.zeros((ZR, D), jnp.float32)

    one_hop, deg = _spmm_deg(x, row, col, z2)
    (two_hop,) = _spmm(one_hop, row, col, z2)

    ef = edges.astype(jnp.int32).reshape(QF)
    gx, g1, g2, gd = _qgather(x, one_hop, two_hop, deg, ef)

    x0, x1 = gx[:Q], gx[Q:]
    h10, h11 = g1[:Q], g1[Q:]
    h20, h21 = g2[:Q], g2[Q:]
    d0 = gd[:Q].reshape(Q, 1)
    d1 = gd[Q:].reshape(Q, 1)

    o11, o12, o22, os12 = _decode(x0, x1, h10, h11, h20, h21, d0, d1)
    return (o11.reshape(Q), o12.reshape(Q), o22.reshape(Q), os12.reshape(Q))
